# Initial kernel scaffold; baseline (speedup 1.0000x reference)
import functools

import jax
import jax.numpy as jnp
from jax import lax
from jax.experimental import pallas as pl
from jax.experimental.pallas import tpu as pltpu
from jax.experimental.pallas import tpu_sc as plsc

N_NODES = 10000
N_EDGES = 320000
D_EDGE = 4

NC = 2
NS = 16
NW = NC * NS
E_PER_W = N_EDGES // NW
GROUPS = E_PER_W // 16

DROWS = 80
DCOLS = 128

_mesh = plsc.VectorSubcoreMesh(core_axis_name="c", subcore_axis_name="s")


def _degrees_body(attr_hbm, eu_hbm, ev_hbm, out_hbm,
                  attr_v, eu_v, ev_v, deg1_v, deg2_v, ridx_v,
                  acc1_sh, acc2_sh, sem):
    cid = lax.axis_index("c")
    sid = lax.axis_index("s")
    wid = sid * NC + cid
    base = wid * E_PER_W

    attr_cp = pltpu.async_copy(attr_hbm.at[pl.ds(base, E_PER_W)], attr_v, sem)
    eu_cp = pltpu.async_copy(eu_hbm.at[pl.ds(base, E_PER_W)], eu_v, sem)
    ev_cp = pltpu.async_copy(ev_hbm.at[pl.ds(base, E_PER_W)], ev_v, sem)

    zeros = jnp.zeros((16,), jnp.float32)

    def _zero_row(r, carry):
        for cc in range(DCOLS // 16):
            deg1_v[r, pl.ds(cc * 16, 16)] = zeros
            deg2_v[r, pl.ds(cc * 16, 16)] = zeros
        return carry

    lax.fori_loop(0, DROWS, _zero_row, 0)

    iota16 = lax.iota(jnp.int32, 16)
    for i in range(DROWS // 16):
        ridx_v[pl.ds(i * 16, 16)] = iota16 + (i * 16)

    @pl.when(sid == 0)
    def _():
        pltpu.sync_copy(deg1_v, acc1_sh)
        pltpu.sync_copy(deg2_v, acc2_sh)

    attr_cp.wait()
    eu_cp.wait()
    ev_cp.wait()

    ones = jnp.ones((16,), jnp.float32)

    def _group(g, carry):
        e0 = g * 16
        ev_idx = iota16 + e0
        a0 = plsc.load_gather(attr_v, [ev_idx, jnp.zeros((16,), jnp.int32)])
        a1 = plsc.load_gather(attr_v, [ev_idx, jnp.full((16,), 1, jnp.int32)])
        a2 = plsc.load_gather(attr_v, [ev_idx, jnp.full((16,), 2, jnp.int32)])
        a3 = plsc.load_gather(attr_v, [ev_idx, jnp.full((16,), 3, jnp.int32)])
        m1 = (a1 > a0) & (a1 >= a2) & (a1 >= a3)
        m2 = (a2 > a0) & (a2 > a1) & (a2 >= a3)
        u = eu_v[pl.ds(e0, 16)]
        v = ev_v[pl.ds(e0, 16)]
        ur = lax.shift_right_logical(u, 7)
        uc = lax.bitwise_and(u, 127)
        vr = lax.shift_right_logical(v, 7)
        vc = lax.bitwise_and(v, 127)
        plsc.addupdate_scatter(deg1_v, [ur, uc], ones, mask=m1)
        plsc.addupdate_scatter(deg1_v, [vr, vc], ones, mask=m1)
        plsc.addupdate_scatter(deg2_v, [ur, uc], ones, mask=m2)
        plsc.addupdate_scatter(deg2_v, [vr, vc], ones, mask=m2)
        return carry

    lax.fori_loop(0, GROUPS, _group, 0)

    plsc.subcore_barrier()
    pltpu.sync_copy(deg1_v, acc1_sh.at[ridx_v], add=True)
    pltpu.sync_copy(deg2_v, acc2_sh.at[ridx_v], add=True)
    plsc.subcore_barrier()

    @pl.when(sid == 0)
    def _():
        pltpu.sync_copy(acc1_sh, out_hbm.at[cid, 0])
        pltpu.sync_copy(acc2_sh, out_hbm.at[cid, 1])


@functools.partial(
    pl.kernel,
    out_type=jax.ShapeDtypeStruct((NC, 2, DROWS, DCOLS), jnp.float32),
    mesh=_mesh,
    scratch_types=[
        pltpu.VMEM((E_PER_W, D_EDGE), jnp.float32),
        pltpu.VMEM((E_PER_W,), jnp.int32),
        pltpu.VMEM((E_PER_W,), jnp.int32),
        pltpu.VMEM((DROWS, DCOLS), jnp.float32),
        pltpu.VMEM((DROWS, DCOLS), jnp.float32),
        pltpu.VMEM((DROWS,), jnp.int32),
        pltpu.VMEM_SHARED((DROWS, DCOLS), jnp.float32),
        pltpu.VMEM_SHARED((DROWS, DCOLS), jnp.float32),
        pltpu.SemaphoreType.DMA,
    ],
)
def _degrees_kernel(*args):
    _degrees_body(*args)


ROWS_PER_TILE = DROWS // 8


def _energy_body(part_hbm, out_hbm, p0_v, p1_v, stage_v, psum_sh, sem):
    cid = lax.axis_index("c")
    sid = lax.axis_index("s")

    @pl.when(cid == 0)
    def _():
        q = lax.bitwise_and(sid, 1)
        r0 = lax.shift_right_logical(sid, 1) * ROWS_PER_TILE
        c0 = pltpu.async_copy(part_hbm.at[0, q, pl.ds(r0, ROWS_PER_TILE)], p0_v, sem)
        c1 = pltpu.async_copy(part_hbm.at[1, q, pl.ds(r0, ROWS_PER_TILE)], p1_v, sem)
        c0.wait()
        c1.wait()
        acc = jnp.zeros((16,), jnp.float32)
        for r in range(ROWS_PER_TILE):
            for cc in range(DCOLS // 16):
                x = p0_v[r, pl.ds(cc * 16, 16)] + p1_v[r, pl.ds(cc * 16, 16)]
                acc = acc + x * x
        stage_v[0, pl.ds(0, 16)] = acc
        pltpu.sync_copy(stage_v.at[0], psum_sh.at[sid])

    plsc.subcore_barrier()

    @pl.when((cid == 0) & (sid == 0))
    def _():
        pltpu.sync_copy(psum_sh, stage_v)
        tot = jnp.zeros((16,), jnp.float32)
        for i in range(NS):
            tot = tot + stage_v[i, pl.ds(0, 16)]
        s = lax.reduce_sum_p.bind(tot, axes=(0,))
        stage_v[0, pl.ds(0, 16)] = jnp.full((16,), s, jnp.float32)
        pltpu.sync_copy(stage_v.at[0], out_hbm)


@functools.partial(
    pl.kernel,
    out_type=jax.ShapeDtypeStruct((16,), jnp.float32),
    mesh=_mesh,
    scratch_types=[
        pltpu.VMEM((ROWS_PER_TILE, DCOLS), jnp.float32),
        pltpu.VMEM((ROWS_PER_TILE, DCOLS), jnp.float32),
        pltpu.VMEM((NS, 16), jnp.float32),
        pltpu.VMEM_SHARED((NS, 16), jnp.float32),
        pltpu.SemaphoreType.DMA,
    ],
)
def _energy_kernel(*args):
    _energy_body(*args)


def kernel(node_features, edge_attr, coupling_strength, edge_index):
    num_nodes = node_features.shape[0]
    esum = _energy_kernel(_degrees_kernel(edge_attr, edge_index[0], edge_index[1]))
    return esum[0] * coupling_strength / (2.0 * num_nodes)


# speedup vs baseline: 5.0030x; 5.0030x over previous
import functools

import jax
import jax.numpy as jnp
from jax import lax
from jax.experimental import pallas as pl
from jax.experimental.pallas import tpu as pltpu
from jax.experimental.pallas import tpu_sc as plsc

N_NODES = 10000
N_EDGES = 320000
D_EDGE = 4

NC = 2
NS = 16
NW = NC * NS
E_PER_W = N_EDGES // NW
GROUPS = E_PER_W // 16

DROWS = 80
DCOLS = 128

_mesh = plsc.VectorSubcoreMesh(
    core_axis_name="c", subcore_axis_name="s", num_cores=NC, num_subcores=NS
)


def _degrees_body(attr_hbm, eu_hbm, ev_hbm, out_hbm,
                  attr_v, eu_v, ev_v, deg1_v, deg2_v, ridx_v,
                  acc1_sh, acc2_sh, sem_a, sem_u, sem_v):
    cid = lax.axis_index("c")
    sid = lax.axis_index("s")
    wid = sid * NC + cid
    base = wid * E_PER_W

    attr_cp = pltpu.async_copy(
        attr_hbm.at[pl.ds(base * D_EDGE, E_PER_W * D_EDGE)], attr_v, sem_a
    )
    eu_cp = pltpu.async_copy(eu_hbm.at[pl.ds(base, E_PER_W)], eu_v, sem_u)
    ev_cp = pltpu.async_copy(ev_hbm.at[pl.ds(base, E_PER_W)], ev_v, sem_v)

    zeros = jnp.zeros((16,), jnp.float32)

    def _zero_row(r, carry):
        for cc in range(DCOLS // 16):
            deg1_v[r, pl.ds(cc * 16, 16)] = zeros
            deg2_v[r, pl.ds(cc * 16, 16)] = zeros
        return carry

    lax.fori_loop(0, DROWS, _zero_row, 0)

    iota16 = lax.iota(jnp.int32, 16)
    for i in range(DROWS // 16):
        ridx_v[pl.ds(i * 16, 16)] = iota16 + (i * 16)

    @pl.when(sid == 0)
    def _():
        pltpu.sync_copy(deg1_v, acc1_sh)
        pltpu.sync_copy(deg2_v, acc2_sh)

    attr_cp.wait()
    eu_cp.wait()
    ev_cp.wait()

    ones = jnp.ones((16,), jnp.float32)

    iota4 = iota16 * D_EDGE

    def _group(g, carry):
        e0 = g * 16
        f0 = iota4 + (e0 * D_EDGE)
        a0 = plsc.load_gather(attr_v, [f0])
        a1 = plsc.load_gather(attr_v, [f0 + 1])
        a2 = plsc.load_gather(attr_v, [f0 + 2])
        a3 = plsc.load_gather(attr_v, [f0 + 3])
        m1 = (a1 > a0) & (a1 >= a2) & (a1 >= a3)
        m2 = (a2 > a0) & (a2 > a1) & (a2 >= a3)
        u = eu_v[pl.ds(e0, 16)]
        v = ev_v[pl.ds(e0, 16)]
        ur = lax.shift_right_logical(u, 7)
        uc = lax.bitwise_and(u, 127)
        vr = lax.shift_right_logical(v, 7)
        vc = lax.bitwise_and(v, 127)
        plsc.addupdate_scatter(deg1_v, [ur, uc], ones, mask=m1)
        plsc.addupdate_scatter(deg1_v, [vr, vc], ones, mask=m1)
        plsc.addupdate_scatter(deg2_v, [ur, uc], ones, mask=m2)
        plsc.addupdate_scatter(deg2_v, [vr, vc], ones, mask=m2)
        return carry

    lax.fori_loop(0, GROUPS, _group, 0)

    plsc.subcore_barrier()
    pltpu.sync_copy(deg1_v, acc1_sh.at[ridx_v], add=True)
    pltpu.sync_copy(deg2_v, acc2_sh.at[ridx_v], add=True)
    plsc.subcore_barrier()

    @pl.when(sid == 0)
    def _():
        pltpu.sync_copy(acc1_sh, out_hbm.at[cid, 0])
        pltpu.sync_copy(acc2_sh, out_hbm.at[cid, 1])


@functools.partial(
    pl.kernel,
    out_type=jax.ShapeDtypeStruct((NC, 2, DROWS, DCOLS), jnp.float32),
    mesh=_mesh,
    scratch_types=[
        pltpu.VMEM((E_PER_W * D_EDGE,), jnp.float32),
        pltpu.VMEM((E_PER_W,), jnp.int32),
        pltpu.VMEM((E_PER_W,), jnp.int32),
        pltpu.VMEM((DROWS, DCOLS), jnp.float32),
        pltpu.VMEM((DROWS, DCOLS), jnp.float32),
        pltpu.VMEM((DROWS,), jnp.int32),
        pltpu.VMEM_SHARED((DROWS, DCOLS), jnp.float32),
        pltpu.VMEM_SHARED((DROWS, DCOLS), jnp.float32),
        pltpu.SemaphoreType.DMA,
        pltpu.SemaphoreType.DMA,
        pltpu.SemaphoreType.DMA,
    ],
    compiler_params=pltpu.CompilerParams(needs_layout_passes=False),
)
def _degrees_kernel(*args):
    _degrees_body(*args)


HALF = 2 * DROWS * DCOLS
CHUNK = HALF // NS
CGROUPS = CHUNK // 16


def _energy_body(part_hbm, out_hbm, p0_v, p1_v, stage_v, acc_sm, sem0, sem1):
    cid = lax.axis_index("c")
    sid = lax.axis_index("s")

    @pl.when((cid == 0) & (sid == 0))
    def _():
        acc_sm[0] = 0

    plsc.subcore_barrier()

    @pl.when(cid == 0)
    def _():
        o = sid * CHUNK
        c0 = pltpu.async_copy(part_hbm.at[pl.ds(o, CHUNK)], p0_v, sem0)
        c1 = pltpu.async_copy(part_hbm.at[pl.ds(HALF + o, CHUNK)], p1_v, sem1)
        c0.wait()
        c1.wait()
        acc = jnp.zeros((16,), jnp.float32)
        for g in range(CGROUPS):
            x = p0_v[pl.ds(g * 16, 16)] + p1_v[pl.ds(g * 16, 16)]
            acc = acc + x * x
        s = lax.reduce_sum_p.bind(acc, axes=(0,))
        plsc.fetch_and_add(acc_sm, s.astype(jnp.int32), subcore_id=0)

    plsc.subcore_barrier()

    @pl.when((cid == 0) & (sid == 0))
    def _():
        tot = acc_sm[0].astype(jnp.float32)
        stage_v[0, pl.ds(0, 16)] = jnp.full((16,), tot, jnp.float32)
        pltpu.sync_copy(stage_v.at[0], out_hbm)


@functools.partial(
    pl.kernel,
    out_type=jax.ShapeDtypeStruct((16,), jnp.float32),
    mesh=_mesh,
    scratch_types=[
        pltpu.VMEM((CHUNK,), jnp.float32),
        pltpu.VMEM((CHUNK,), jnp.float32),
        pltpu.VMEM((1, 16), jnp.float32),
        pltpu.SMEM((1,), jnp.int32),
        pltpu.SemaphoreType.DMA,
        pltpu.SemaphoreType.DMA,
    ],
    compiler_params=pltpu.CompilerParams(needs_layout_passes=False),
)
def _energy_kernel(*args):
    _energy_body(*args)


def kernel(node_features, edge_attr, coupling_strength, edge_index):
    num_nodes = node_features.shape[0]
    part = _degrees_kernel(edge_attr.reshape(-1), edge_index[0], edge_index[1])
    esum = _energy_kernel(part.reshape(-1))
    return esum[0] * coupling_strength / (2.0 * num_nodes)
